# baseline (speedup 1.0000x reference)
import functools

import jax
import jax.numpy as jnp
from jax import lax
from jax.experimental import pallas as pl
from jax.experimental.pallas import tpu as pltpu
from jax.experimental.pallas import tpu_sc as plsc

B = 16384
D = 128
H = 1024
KE = 2 * D + 8

NCHUNK = 4
BC = B // NCHUNK

_INFO = plsc.get_sparse_core_info()
_NC, _NS = _INFO.num_cores, _INFO.num_subcores
_NW = _NC * _NS
_BPW = BC // _NW
_CH = 128
_NG = _BPW // _CH

_sc_mesh = plsc.VectorSubcoreMesh(core_axis_name="c", subcore_axis_name="s")


@functools.partial(
    pl.kernel,
    mesh=_sc_mesh,
    out_type=[
        jax.ShapeDtypeStruct((BC, D), jnp.float32),
        jax.ShapeDtypeStruct((BC, D), jnp.float32),
    ],
    scratch_types=[
        pltpu.VMEM((_NG, _CH), jnp.int32),
        pltpu.VMEM((_NG, _CH), jnp.int32),
        pltpu.VMEM((_BPW, D), jnp.float32),
        pltpu.VMEM((_BPW, D), jnp.float32),
        pltpu.SemaphoreType.DMA,
    ],
)
def _sc_gather(xu_hbm, xm_hbm, ut_hbm, mt_hbm, u_out, m_out, idxu_v, idxm_v,
               urows_v, mrows_v, sem):
    wid = lax.axis_index("s") * _NC + lax.axis_index("c")
    base = wid * _BPW
    cbase = wid * _NG

    pltpu.sync_copy(xu_hbm.at[pl.ds(cbase, _NG)], idxu_v)
    pltpu.sync_copy(xm_hbm.at[pl.ds(cbase, _NG)], idxm_v)
    cps = [
        pltpu.async_copy(
            ut_hbm.at[idxu_v.at[j]], urows_v.at[pl.ds(j * _CH, _CH)], sem
        )
        for j in range(_NG)
    ] + [
        pltpu.async_copy(
            mt_hbm.at[idxm_v.at[j]], mrows_v.at[pl.ds(j * _CH, _CH)], sem
        )
        for j in range(_NG)
    ]
    for cp in cps:
        cp.wait()
    pltpu.sync_copy(urows_v, u_out.at[pl.ds(base, _BPW)])
    pltpu.sync_copy(mrows_v, m_out.at[pl.ds(base, _BPW)])


_BB = 2048


def _mlp_body(u_ref, m_ref, w1e_ref, w2_ref, b2_ref, acc_ref, out_ref):
    del acc_ref
    ut = u_ref[...].astype(jnp.bfloat16).T
    mt = m_ref[...].astype(jnp.bfloat16).T
    xt = jnp.concatenate(
        [ut, mt, jnp.ones((8, _BB), jnp.bfloat16)], axis=0
    )
    hT = jnp.dot(w1e_ref[...], xt, preferred_element_type=jnp.float32)
    hT = jnp.maximum(hT, 0.0)
    g = jnp.dot(w2_ref[...], hT, preferred_element_type=jnp.float32)
    out_ref[...] = g.reshape(_BB // 128, 128) + b2_ref[0, 0]


def _mlp_chunk(c, u, m, w1e, w2r, b2r, acc):
    cbase = c * (BC // _BB)
    return pl.pallas_call(
        _mlp_body,
        grid=(BC // _BB,),
        in_specs=[
            pl.BlockSpec((_BB, D), lambda i: (i, 0)),
            pl.BlockSpec((_BB, D), lambda i: (i, 0)),
            pl.BlockSpec((H, KE), lambda i: (0, 0)),
            pl.BlockSpec((1, H), lambda i: (0, 0)),
            pl.BlockSpec((1, 1), lambda i: (0, 0)),
            pl.BlockSpec(memory_space=pl.ANY),
        ],
        out_specs=pl.BlockSpec(
            (_BB // 128, 128), lambda i, cbase=cbase: (cbase + i, 0)
        ),
        out_shape=jax.ShapeDtypeStruct((B // 128, 128), jnp.float32),
        input_output_aliases={5: 0},
    )(u, m, w1e, w2r, b2r, acc)


def kernel(X, user_table, movie_table, W1, b1, W2, b2):
    xu = X[:, 0].astype(jnp.int32).reshape(B // _CH, _CH)
    xm = X[:, 1].astype(jnp.int32).reshape(B // _CH, _CH)
    w1e = jnp.concatenate(
        [W1.T, b1.reshape(H, 1), jnp.zeros((H, 7), jnp.float32)], axis=1
    ).astype(jnp.bfloat16)
    w2r = W2.reshape(1, H)
    b2r = b2.reshape(1, 1)

    rows_per_chunk = BC // _CH
    acc = jnp.zeros((B // 128, 128), jnp.float32)
    for c in range(NCHUNK):
        s = c * rows_per_chunk
        u_c, m_c = _sc_gather(
            xu[s:s + rows_per_chunk],
            xm[s:s + rows_per_chunk],
            user_table,
            movie_table,
        )
        acc = _mlp_chunk(c, u_c, m_c, w1e, w2r, b2r, acc)
    return acc.reshape(B, 1)


# speedup vs baseline: 1.0783x; 1.0783x over previous
import functools

import jax
import jax.numpy as jnp
from jax import lax
from jax.experimental import pallas as pl
from jax.experimental.pallas import tpu as pltpu
from jax.experimental.pallas import tpu_sc as plsc

B = 16384
D = 128
H = 1024
KE = 2 * D + 8

NCHUNK = 2
BC = B // NCHUNK

_INFO = plsc.get_sparse_core_info()
_NC, _NS = _INFO.num_cores, _INFO.num_subcores
_NW = _NC * _NS
_BPW = BC // _NW
_CH = 128
_NG = _BPW // _CH

_sc_mesh = plsc.VectorSubcoreMesh(core_axis_name="c", subcore_axis_name="s")


@functools.partial(
    pl.kernel,
    mesh=_sc_mesh,
    out_type=[
        jax.ShapeDtypeStruct((BC, D), jnp.float32),
        jax.ShapeDtypeStruct((BC, D), jnp.float32),
    ],
    scratch_types=[
        pltpu.VMEM((_NG, _CH), jnp.int32),
        pltpu.VMEM((_NG, _CH), jnp.int32),
        pltpu.VMEM((_BPW, D), jnp.float32),
        pltpu.VMEM((_BPW, D), jnp.float32),
        pltpu.SemaphoreType.DMA,
    ],
)
def _sc_gather(xu_hbm, xm_hbm, ut_hbm, mt_hbm, u_out, m_out, idxu_v, idxm_v,
               urows_v, mrows_v, sem):
    wid = lax.axis_index("s") * _NC + lax.axis_index("c")
    base = wid * _BPW
    cbase = wid * _NG

    pltpu.sync_copy(xu_hbm.at[pl.ds(cbase, _NG)], idxu_v)
    pltpu.sync_copy(xm_hbm.at[pl.ds(cbase, _NG)], idxm_v)
    cps = [
        pltpu.async_copy(
            ut_hbm.at[idxu_v.at[j]], urows_v.at[pl.ds(j * _CH, _CH)], sem
        )
        for j in range(_NG)
    ] + [
        pltpu.async_copy(
            mt_hbm.at[idxm_v.at[j]], mrows_v.at[pl.ds(j * _CH, _CH)], sem
        )
        for j in range(_NG)
    ]
    for cp in cps:
        cp.wait()
    pltpu.sync_copy(urows_v, u_out.at[pl.ds(base, _BPW)])
    pltpu.sync_copy(mrows_v, m_out.at[pl.ds(base, _BPW)])


_BB = 2048


def _mlp_body(u_ref, m_ref, w1ut_ref, w1mt_ref, b1_ref, w2_ref, b2_ref,
              acc_ref, out_ref):
    del acc_ref
    ut = u_ref[...].astype(jnp.bfloat16).T
    mt = m_ref[...].astype(jnp.bfloat16).T
    hT = (
        jnp.dot(w1ut_ref[...], ut, preferred_element_type=jnp.float32)
        + jnp.dot(w1mt_ref[...], mt, preferred_element_type=jnp.float32)
        + b1_ref[...]
    )
    hT = jnp.maximum(hT, 0.0)
    g = jnp.dot(w2_ref[...], hT, preferred_element_type=jnp.float32)
    out_ref[...] = g.reshape(_BB // 128, 128) + b2_ref[0, 0]


def _mlp_chunk(c, u, m, w1ut, w1mt, b1r, w2r, b2r, acc):
    cbase = c * (BC // _BB)
    return pl.pallas_call(
        _mlp_body,
        grid=(BC // _BB,),
        in_specs=[
            pl.BlockSpec((_BB, D), lambda i: (i, 0)),
            pl.BlockSpec((_BB, D), lambda i: (i, 0)),
            pl.BlockSpec((H, D), lambda i: (0, 0)),
            pl.BlockSpec((H, D), lambda i: (0, 0)),
            pl.BlockSpec((H, 1), lambda i: (0, 0)),
            pl.BlockSpec((1, H), lambda i: (0, 0)),
            pl.BlockSpec((1, 1), lambda i: (0, 0)),
            pl.BlockSpec(memory_space=pl.ANY),
        ],
        out_specs=pl.BlockSpec(
            (_BB // 128, 128), lambda i, cbase=cbase: (cbase + i, 0)
        ),
        out_shape=jax.ShapeDtypeStruct((B // 128, 128), jnp.float32),
        input_output_aliases={7: 0},
    )(u, m, w1ut, w1mt, b1r, w2r, b2r, acc)


def kernel(X, user_table, movie_table, W1, b1, W2, b2):
    xu = X[:, 0].astype(jnp.int32).reshape(B // _CH, _CH)
    xm = X[:, 1].astype(jnp.int32).reshape(B // _CH, _CH)
    w1ut = W1[:D].T.astype(jnp.bfloat16)
    w1mt = W1[D:].T.astype(jnp.bfloat16)
    b1r = b1.reshape(H, 1)
    w2r = W2.reshape(1, H)
    b2r = b2.reshape(1, 1)

    rows_per_chunk = BC // _CH
    acc = jnp.zeros((B // 128, 128), jnp.float32)
    for c in range(NCHUNK):
        s = c * rows_per_chunk
        u_c, m_c = _sc_gather(
            xu[s:s + rows_per_chunk],
            xm[s:s + rows_per_chunk],
            user_table,
            movie_table,
        )
        acc = _mlp_chunk(c, u_c, m_c, w1ut, w1mt, b1r, w2r, b2r, acc)
    return acc.reshape(B, 1)
